# baseline (speedup 1.0000x reference)
import functools

import jax
import jax.numpy as jnp
from jax import lax
from jax.experimental import pallas as pl
from jax.experimental.pallas import tpu as pltpu
from jax.experimental.pallas import tpu_sc as plsc

D = 64
B = 4096 * 20
NC, NS = 2, 16
NW = NC * NS
BPW = B // NW
CHUNK = 128
NCH = BPW // CHUNK

_mesh = plsc.VectorSubcoreMesh(core_axis_name="c", subcore_axis_name="s")


@functools.partial(
    pl.kernel,
    mesh=_mesh,
    out_type=jax.ShapeDtypeStruct((B, D), jnp.float32),
    compiler_params=pltpu.CompilerParams(use_tc_tiling_on_sc=False),
    scratch_types=[
        pltpu.VMEM((NCH, CHUNK), jnp.int32),
        pltpu.VMEM((CHUNK, D), jnp.float32),
        pltpu.VMEM((CHUNK, D), jnp.float32),
        pltpu.SemaphoreType.DMA,
        pltpu.SemaphoreType.DMA,
    ],
)
def _emb_lookup(idx_hbm, table_hbm, out_hbm, idx_v, buf0, buf1, sem0, sem1):
    wid = lax.axis_index("s") * NC + lax.axis_index("c")
    base = wid * BPW
    pltpu.sync_copy(idx_hbm.at[wid], idx_v)
    bufs = (buf0, buf1)
    sems = (sem0, sem1)
    handles = [
        pltpu.async_copy(table_hbm.at[idx_v.at[0]], buf0, sem0),
        pltpu.async_copy(table_hbm.at[idx_v.at[1]], buf1, sem1),
    ]
    for ch in range(NCH):
        p = ch % 2
        handles[p].wait()
        pltpu.sync_copy(bufs[p], out_hbm.at[pl.ds(base + ch * CHUNK, CHUNK)])
        if ch + 2 < NCH:
            handles[p] = pltpu.async_copy(
                table_hbm.at[idx_v.at[ch + 2]], bufs[p], sems[p]
            )


def kernel(token_ids, embedding):
    idx = token_ids.astype(jnp.int32).reshape(NW, NCH, CHUNK)
    out = _emb_lookup(idx, embedding)
    return out.reshape(*token_ids.shape, D)


# speedup vs baseline: 1.5358x; 1.5358x over previous
import functools

import jax
import jax.numpy as jnp
from jax import lax
from jax.experimental import pallas as pl
from jax.experimental.pallas import tpu as pltpu
from jax.experimental.pallas import tpu_sc as plsc

V = 1000
D = 64
J = 20
B = 4096
NC, NS = 2, 16
NW = NC * NS
BQ = B // NW
L = 16

_mesh = plsc.VectorSubcoreMesh(core_axis_name="c", subcore_axis_name="s")


@functools.partial(
    pl.kernel,
    mesh=_mesh,
    out_type=jax.ShapeDtypeStruct((J, D, B), jnp.float32),
    compiler_params=pltpu.CompilerParams(needs_layout_passes=False),
    scratch_types=[
        pltpu.VMEM((J, BQ), jnp.int32),
        pltpu.VMEM((V * D,), jnp.float32),
        pltpu.VMEM((D, BQ), jnp.float32),
        pltpu.VMEM((D, BQ), jnp.float32),
        pltpu.SemaphoreType.DMA,
        pltpu.SemaphoreType.DMA,
    ],
)
def _emb_lookup(tok_hbm, tab_hbm, out_hbm, tok_v, tab_v, buf0, buf1, s0, s1):
    wid = lax.axis_index("s") * NC + lax.axis_index("c")
    b0 = wid * BQ
    pltpu.sync_copy(tok_hbm.at[:, wid], tok_v)
    pltpu.sync_copy(tab_hbm, tab_v)

    bufs = (buf0, buf1)
    sems = (s0, s1)

    def fill(j, buf):
        def fi(i, _):
            base = i * L
            idx = tok_v[j, pl.ds(base, L)]
            for d in range(D):
                buf[d, pl.ds(base, L)] = plsc.load_gather(tab_v, [idx + d * V])
            return 0

        lax.fori_loop(0, BQ // L, fi, 0)

    def wait_write(p):
        pltpu.make_async_copy(
            bufs[p], out_hbm.at[0, :, pl.ds(b0, BQ)], sems[p]
        ).wait()

    def body(jj, _):
        for p in range(2):
            j = 2 * jj + p

            @pl.when(jj > 0)
            def _():
                wait_write(p)

            fill(j, bufs[p])
            pltpu.async_copy(bufs[p], out_hbm.at[j, :, pl.ds(b0, BQ)], sems[p])
        return 0

    lax.fori_loop(0, J // 2, body, 0)
    wait_write(0)
    wait_write(1)


def kernel(token_ids, embedding):
    tok = token_ids.astype(jnp.int32).T.reshape(J, NW, BQ)
    tab = embedding.T.reshape(-1)
    out = _emb_lookup(tok, tab)
    return out.transpose(2, 0, 1)
